# baseline (speedup 1.0000x reference)
import functools

import jax
import jax.numpy as jnp
from jax import lax
from jax.experimental import pallas as pl
from jax.experimental.pallas import tpu as pltpu
from jax.experimental.pallas import tpu_sc as plsc

B = 1024
N = 200
K = 50
D = 128

NC = 2
NS = 16
NW = NC * NS
ROWS = B * K
RPW = ROWS // NW
CH = 64
NCH = RPW // CH
NBUF = 15


def _build_sc_gather():
    mesh = plsc.VectorSubcoreMesh(core_axis_name="c", subcore_axis_name="s")

    @functools.partial(
        pl.kernel,
        mesh=mesh,
        out_type=jax.ShapeDtypeStruct((K * B, D), jnp.float32),
        scratch_types=[
            pltpu.VMEM((NCH, CH), jnp.int32),
        ] + [pltpu.VMEM((CH, D), jnp.float32) for _ in range(NBUF)]
          + [pltpu.SemaphoreType.DMA for _ in range(2 * NBUF)],
    )
    def sc_gather(table_hbm, gid_hbm, out_hbm, gid_v, *bufs_and_sems):
        bufs = bufs_and_sems[:NBUF]
        gsems = bufs_and_sems[NBUF:2 * NBUF]
        wsems = bufs_and_sems[2 * NBUF:]
        wid = lax.axis_index("s") * NC + lax.axis_index("c")
        base = wid * RPW

        pltpu.sync_copy(gid_hbm.at[wid], gid_v)

        gh = [None] * NBUF
        wh = [None] * NBUF
        for j in range(min(NBUF, NCH)):
            gh[j] = pltpu.async_copy(
                table_hbm.at[gid_v.at[j]], bufs[j], gsems[j])
        for j in range(NCH):
            b = j % NBUF
            gh[b].wait()
            wh[b] = pltpu.async_copy(
                bufs[b], out_hbm.at[pl.ds(base + j * CH, CH)], wsems[b])
            nj = j + NBUF
            if nj < NCH:
                wh[b].wait()
                gh[b] = pltpu.async_copy(
                    table_hbm.at[gid_v.at[nj]], bufs[b], gsems[b])
        for j in range(max(0, NCH - NBUF), NCH):
            wh[j % NBUF].wait()

    return sc_gather


_sc_gather = _build_sc_gather()


def kernel(input_tensor, indices):
    table = input_tensor.reshape(B * N, D)
    idx = indices.astype(jnp.int32)
    off = (jnp.arange(B, dtype=jnp.int32) * N)[:, None]
    gid = (idx + off).T.reshape(NW, NCH, CH)
    out = _sc_gather(table, gid)
    return out.reshape(K, B, D).transpose(1, 0, 2)


# speedup vs baseline: 1.0147x; 1.0147x over previous
import functools

import jax
import jax.numpy as jnp
from jax import lax
from jax.experimental import pallas as pl
from jax.experimental.pallas import tpu as pltpu
from jax.experimental.pallas import tpu_sc as plsc

B = 1024
N = 200
K = 50
D = 128

NC = 2
NS = 16
NW = NC * NS
ROWS = B * K
RPW = ROWS // NW
CH = 80
NCH = RPW // CH
NBUF = 12


def _build_sc_gather():
    mesh = plsc.VectorSubcoreMesh(core_axis_name="c", subcore_axis_name="s")

    @functools.partial(
        pl.kernel,
        mesh=mesh,
        out_type=jax.ShapeDtypeStruct((K * B, D), jnp.float32),
        scratch_types=[
            pltpu.VMEM((NCH, CH), jnp.int32),
        ] + [pltpu.VMEM((CH, D), jnp.float32) for _ in range(NBUF)]
          + [pltpu.SemaphoreType.DMA for _ in range(2 * NBUF)],
    )
    def sc_gather(table_hbm, gid_hbm, out_hbm, gid_v, *bufs_and_sems):
        bufs = bufs_and_sems[:NBUF]
        gsems = bufs_and_sems[NBUF:2 * NBUF]
        wsems = bufs_and_sems[2 * NBUF:]
        wid = lax.axis_index("s") * NC + lax.axis_index("c")
        base = wid * RPW

        pltpu.sync_copy(gid_hbm.at[wid], gid_v)

        gh = [None] * NBUF
        wh = [None] * NBUF
        for j in range(min(NBUF, NCH)):
            gh[j] = pltpu.async_copy(
                table_hbm.at[gid_v.at[j]], bufs[j], gsems[j])
        for j in range(NCH):
            b = j % NBUF
            gh[b].wait()
            wh[b] = pltpu.async_copy(
                bufs[b], out_hbm.at[pl.ds(base + j * CH, CH)], wsems[b])
            nj = j + NBUF
            if nj < NCH:
                wh[b].wait()
                gh[b] = pltpu.async_copy(
                    table_hbm.at[gid_v.at[nj]], bufs[b], gsems[b])
        for j in range(max(0, NCH - NBUF), NCH):
            wh[j % NBUF].wait()

    return sc_gather


_sc_gather = _build_sc_gather()


def kernel(input_tensor, indices):
    table = input_tensor.reshape(B * N, D)
    idx = indices.astype(jnp.int32)
    off = (jnp.arange(B, dtype=jnp.int32) * N)[:, None]
    gid = (idx + off).T.reshape(NW, NCH, CH)
    out = _sc_gather(table, gid)
    return out.reshape(K, B, D).transpose(1, 0, 2)


# speedup vs baseline: 1.0248x; 1.0099x over previous
import functools

import jax
import jax.numpy as jnp
from jax import lax
from jax.experimental import pallas as pl
from jax.experimental.pallas import tpu as pltpu
from jax.experimental.pallas import tpu_sc as plsc

B = 1024
N = 200
K = 50
D = 128

NC = 2
NS = 16
NW = NC * NS
ROWS = B * K
RPW = ROWS // NW
CH = 128
NCHF = RPW // CH
TAIL = RPW - NCHF * CH
NCH = NCHF + 1
NBUF = 7


def _build_sc_gather():
    mesh = plsc.VectorSubcoreMesh(core_axis_name="c", subcore_axis_name="s")

    @functools.partial(
        pl.kernel,
        mesh=mesh,
        out_type=jax.ShapeDtypeStruct((K * B, D), jnp.float32),
        scratch_types=[
            pltpu.VMEM((NCH, CH), jnp.int32),
        ] + [pltpu.VMEM((CH, D), jnp.float32) for _ in range(NBUF)]
          + [pltpu.SemaphoreType.DMA for _ in range(2 * NBUF)],
    )
    def sc_gather(table_hbm, gid_hbm, out_hbm, gid_v, *bufs_and_sems):
        bufs = bufs_and_sems[:NBUF]
        gsems = bufs_and_sems[NBUF:2 * NBUF]
        wsems = bufs_and_sems[2 * NBUF:]
        wid = lax.axis_index("s") * NC + lax.axis_index("c")
        base = wid * RPW

        pltpu.sync_copy(gid_hbm.at[wid], gid_v)

        gh = [None] * NBUF
        wh = [None] * NBUF
        for j in range(min(NBUF, NCH)):
            gh[j] = pltpu.async_copy(
                table_hbm.at[gid_v.at[j]], bufs[j], gsems[j])
        for j in range(NCH):
            b = j % NBUF
            wlen = CH if j < NCHF else TAIL
            gh[b].wait()
            wh[b] = pltpu.async_copy(
                bufs[b].at[pl.ds(0, wlen)],
                out_hbm.at[pl.ds(base + j * CH, wlen)], wsems[b])
            nj = j + NBUF
            if nj < NCH:
                wh[b].wait()
                gh[b] = pltpu.async_copy(
                    table_hbm.at[gid_v.at[nj]], bufs[b], gsems[b])
        for j in range(max(0, NCH - NBUF), NCH):
            wh[j % NBUF].wait()

    return sc_gather


_sc_gather = _build_sc_gather()


def kernel(input_tensor, indices):
    table = input_tensor.reshape(B * N, D)
    idx = indices.astype(jnp.int32)
    off = (jnp.arange(B, dtype=jnp.int32) * N)[:, None]
    ids = (idx + off).T.reshape(NW, RPW)
    tail = ids[:, NCHF * CH:]
    gid = jnp.concatenate(
        [ids[:, :NCHF * CH].reshape(NW, NCHF, CH),
         jnp.concatenate([tail, tail], axis=1)[:, None, :]], axis=1)
    out = _sc_gather(table, gid)
    return out.reshape(K, B, D).transpose(1, 0, 2)
